# baseline (speedup 1.0000x reference)
import jax
import jax.numpy as jnp
from jax.experimental import pallas as pl
from jax.experimental.pallas import tpu as pltpu

_BS = 2048


def _ln_body(x_ref, p_ref, g_ref, b_ref, o_ref):
    emb = x_ref[...] + p_ref[...]
    mean = jnp.mean(emb, axis=-1, keepdims=True)
    d = emb - mean
    var = jnp.mean(d * d, axis=-1, keepdims=True)
    o_ref[...] = d * jax.lax.rsqrt(var + 1e-5) * g_ref[...] + b_ref[...]


def kernel(x, pos_table, ln_gamma, ln_beta):
    B, S, D = x.shape
    rows = B * S
    x2 = x.reshape(rows, D)
    g2 = ln_gamma.reshape(1, D)
    b2 = ln_beta.reshape(1, D)
    n_pos_blocks = S // _BS

    out = pl.pallas_call(
        _ln_body,
        grid=(n_pos_blocks, B),
        in_specs=[
            pl.BlockSpec((_BS, D), lambda s, b: (b * n_pos_blocks + s, 0)),
            pl.BlockSpec((_BS, D), lambda s, b: (s, 0)),
            pl.BlockSpec((1, D), lambda s, b: (0, 0)),
            pl.BlockSpec((1, D), lambda s, b: (0, 0)),
        ],
        out_specs=pl.BlockSpec((_BS, D), lambda s, b: (b * n_pos_blocks + s, 0)),
        out_shape=jax.ShapeDtypeStruct((rows, D), x.dtype),
        compiler_params=pltpu.CompilerParams(
            dimension_semantics=("parallel", "parallel"),
        ),
    )(x2, pos_table, g2, b2)
    return out.reshape(B, S, D)


# speedup vs baseline: 1.0519x; 1.0519x over previous
import jax
import jax.numpy as jnp
from jax.experimental import pallas as pl
from jax.experimental.pallas import tpu as pltpu

_BS = 2048


def _ln_body(x_ref, p_ref, g_ref, b_ref, o_ref, *, n_pos_blocks):
    i = pl.program_id(0)
    s = jax.lax.rem(i, n_pos_blocks)
    emb = x_ref[...] + p_ref[pl.ds(s * _BS, _BS), :]
    mean = jnp.mean(emb, axis=-1, keepdims=True)
    d = emb - mean
    var = jnp.mean(d * d, axis=-1, keepdims=True)
    o_ref[...] = d * jax.lax.rsqrt(var + 1e-5) * g_ref[...] + b_ref[...]


def kernel(x, pos_table, ln_gamma, ln_beta):
    import functools
    B, S, D = x.shape
    rows = B * S
    x2 = x.reshape(rows, D)
    g2 = ln_gamma.reshape(1, D)
    b2 = ln_beta.reshape(1, D)
    n_pos_blocks = S // _BS

    out = pl.pallas_call(
        functools.partial(_ln_body, n_pos_blocks=n_pos_blocks),
        grid=(rows // _BS,),
        in_specs=[
            pl.BlockSpec((_BS, D), lambda i: (i, 0)),
            pl.BlockSpec((S, D), lambda i: (0, 0)),
            pl.BlockSpec((1, D), lambda i: (0, 0)),
            pl.BlockSpec((1, D), lambda i: (0, 0)),
        ],
        out_specs=pl.BlockSpec((_BS, D), lambda i: (i, 0)),
        out_shape=jax.ShapeDtypeStruct((rows, D), x.dtype),
        compiler_params=pltpu.CompilerParams(
            dimension_semantics=("arbitrary",),
        ),
    )(x2, pos_table, g2, b2)
    return out.reshape(B, S, D)


# speedup vs baseline: 1.0522x; 1.0003x over previous
import jax
import jax.numpy as jnp
from jax.experimental import pallas as pl
from jax.experimental.pallas import tpu as pltpu

_BS = 2048


def _ln_body(x_ref, p_ref, g_ref, b_ref, o_ref, *, n_pos_blocks):
    i = pl.program_id(0)
    s = jax.lax.rem(i, n_pos_blocks)
    emb = x_ref[...] + p_ref[pl.ds(s * _BS, _BS), :]
    mean = jnp.mean(emb, axis=-1, keepdims=True)
    d = emb - mean
    var = jnp.mean(d * d, axis=-1, keepdims=True)
    o_ref[...] = d * jax.lax.rsqrt(var + 1e-5) * g_ref[...] + b_ref[...]


def kernel(x, pos_table, ln_gamma, ln_beta):
    import functools
    B, S, D = x.shape
    rows = B * S
    x2 = x.reshape(rows, D)
    g2 = ln_gamma.reshape(1, D)
    b2 = ln_beta.reshape(1, D)
    n_pos_blocks = S // _BS

    out = pl.pallas_call(
        functools.partial(_ln_body, n_pos_blocks=n_pos_blocks),
        grid=(rows // _BS,),
        in_specs=[
            pl.BlockSpec((_BS, D), lambda i: (i, 0)),
            pl.BlockSpec((S, D), lambda i: (0, 0)),
            pl.BlockSpec((1, D), lambda i: (0, 0)),
            pl.BlockSpec((1, D), lambda i: (0, 0)),
        ],
        out_specs=pl.BlockSpec((_BS, D), lambda i: (i, 0)),
        out_shape=jax.ShapeDtypeStruct((rows, D), x.dtype),
        compiler_params=pltpu.CompilerParams(
            dimension_semantics=("parallel",),
        ),
    )(x2, pos_table, g2, b2)
    return out.reshape(B, S, D)
